# baseline (speedup 1.0000x reference)
import functools

import jax
import jax.numpy as jnp
from jax import lax
from jax.experimental import pallas as pl
from jax.experimental.pallas import tpu as pltpu
from jax.experimental.pallas import tpu_sc as plsc

_N_USERS = 25000
_N_ITEMS = 25000
_N_NODES = 50000
_E = 800000
_DIM = 64
_BATCH = 1024

_Q = 12544
_ACC_ROWS = 12608
_DUMMY = 12600
_CHUNK = 3200
_NCHT = _E // _CHUNK
_QSTRIDE = 26624
_LISTS = 32 * 4 * _QSTRIDE
_S = 512

_mesh = plsc.VectorSubcoreMesh(core_axis_name="c", subcore_axis_name="s")


def _prefix16(x, iota):
    for d in (1, 2, 4, 8):
        sh = x[jnp.maximum(iota - d, 0)]
        x = x + jnp.where(iota >= d, sh, 0)
    return x


@functools.partial(
    pl.kernel,
    mesh=_mesh,
    compiler_params=pltpu.CompilerParams(use_tc_tiling_on_sc=False),
    out_type=(
        jax.ShapeDtypeStruct((_LISTS,), jnp.int32),
        jax.ShapeDtypeStruct((_LISTS,), jnp.int32),
        jax.ShapeDtypeStruct((_LISTS,), jnp.float32),
        jax.ShapeDtypeStruct((4096,), jnp.int32),
    ),
    scratch_types=[
        pltpu.VMEM((_CHUNK,), jnp.int32),
        pltpu.VMEM((_CHUNK,), jnp.int32),
        pltpu.VMEM((_CHUNK,), jnp.float32),
        pltpu.VMEM((_CHUNK,), jnp.int32),
        pltpu.VMEM((25, 128), jnp.int32),
        pltpu.VMEM((8, 128), jnp.int32),
        pltpu.VMEM((128,), jnp.int32),
        pltpu.VMEM((128,), jnp.int32),
        pltpu.VMEM((128,), jnp.float32),
        pltpu.SemaphoreType.DMA,
    ],
)
def _partition_k(src_hbm, dst_hbm, w_hbm,
                 srcl_hbm, dstl_hbm, wl_hbm, counts_hbm,
                 src_scan, dst_scan, w_scan, ldst_scan,
                 pos_buf, tpos_buf, zi_buf, di_buf, zf_buf, sem):
    c = lax.axis_index("c")
    s = lax.axis_index("s")
    w = s * 2 + c
    iota = lax.iota(jnp.int32, 16)
    base0 = w * (4 * _QSTRIDE)

    nch = jnp.where(w < _NCHT - (_NCHT // 32) * 32, _NCHT // 32 + 1, _NCHT // 32)

    def chunk_body(r, cnts):
        ch = r * 32 + w
        eoff = pl.multiple_of(ch * _CHUNK, 128)
        pltpu.sync_copy(dst_hbm.at[pl.ds(eoff, _CHUNK)], dst_scan)
        pltpu.sync_copy(src_hbm.at[pl.ds(eoff, _CHUNK)], src_scan)
        pltpu.sync_copy(w_hbm.at[pl.ds(eoff, _CHUNK)], w_scan)

        def vec_body(ii, u, cnts):
            c0, c1, c2, c3 = cnts
            boff = ii * 128 + u * 16
            v = dst_scan[pl.ds(boff, 16)]
            m1 = (v >= _Q) & (v < 2 * _Q)
            m2 = (v >= 2 * _Q) & (v < 3 * _Q)
            m3 = v >= 3 * _Q
            p0 = _prefix16(jnp.where(v < _Q, 1, 0), iota)
            p1 = _prefix16(jnp.where(m1, 1, 0), iota)
            p2 = _prefix16(jnp.where(m2, 1, 0), iota)
            p3 = _prefix16(jnp.where(m3, 1, 0), iota)
            pos = base0 + c0 + p0 - 1
            pos = jnp.where(m1, base0 + _QSTRIDE + c1 + p1 - 1, pos)
            pos = jnp.where(m2, base0 + 2 * _QSTRIDE + c2 + p2 - 1, pos)
            pos = jnp.where(m3, base0 + 3 * _QSTRIDE + c3 + p3 - 1, pos)
            pos_buf[ii, pl.ds(u * 16, 16)] = pos
            qoff = jnp.where(m1, _Q, 0)
            qoff = jnp.where(m2, 2 * _Q, qoff)
            qoff = jnp.where(m3, 3 * _Q, qoff)
            ldst_scan[pl.ds(boff, 16)] = v - qoff
            return (c0 + p0[15], c1 + p1[15], c2 + p2[15], c3 + p3[15])

        def oct_body(ii, cnts):
            for u in range(8):
                cnts = vec_body(ii, u, cnts)
            return cnts
        cnts = lax.fori_loop(0, _CHUNK // 128, oct_body, cnts)
        descs = []
        for j in range(25):
            idx = pos_buf.at[j]
            descs.append(pltpu.async_copy(
                src_scan.at[pl.ds(j * 128, 128)], srcl_hbm.at[idx], sem))
            descs.append(pltpu.async_copy(
                ldst_scan.at[pl.ds(j * 128, 128)], dstl_hbm.at[idx], sem))
            descs.append(pltpu.async_copy(
                w_scan.at[pl.ds(j * 128, 128)], wl_hbm.at[idx], sem))
        for d in descs:
            d.wait()
        return cnts

    z = jnp.int32(0)
    cn = lax.fori_loop(0, nch, chunk_body, (z, z, z, z))

    for u in range(8):
        zi_buf[pl.ds(u * 16, 16)] = jnp.zeros((16,), jnp.int32)
        di_buf[pl.ds(u * 16, 16)] = jnp.broadcast_to(_DUMMY, (16,)).astype(jnp.int32)
        zf_buf[pl.ds(u * 16, 16)] = jnp.zeros((16,), jnp.float32)
    for qi in range(4):
        tb = base0 + qi * _QSTRIDE + cn[qi]
        for j in range(8):
            for u in range(8):
                tpos_buf[j, pl.ds(u * 16, 16)] = tb + j * 128 + u * 16 + iota
        descs = []
        for j in range(8):
            idx = tpos_buf.at[j]
            descs.append(pltpu.async_copy(zi_buf, srcl_hbm.at[idx], sem))
            descs.append(pltpu.async_copy(di_buf, dstl_hbm.at[idx], sem))
            descs.append(pltpu.async_copy(zf_buf, wl_hbm.at[idx], sem))
        for d in descs:
            d.wait()

    cv = jnp.where(iota == 0, cn[0], jnp.where(iota == 1, cn[1],
                   jnp.where(iota == 2, cn[2], jnp.where(iota == 3, cn[3], 0))))
    zi_buf[pl.ds(0, 16)] = cv
    for u in range(1, 8):
        zi_buf[pl.ds(u * 16, 16)] = jnp.zeros((16,), jnp.int32)
    pltpu.sync_copy(zi_buf, counts_hbm.at[pl.ds(pl.multiple_of(w * 128, 128), 128)])


def _make_spmm():
    @functools.partial(
        pl.kernel,
        mesh=_mesh,
        compiler_params=pltpu.CompilerParams(use_tc_tiling_on_sc=False),
        out_type=jax.ShapeDtypeStruct((_N_NODES, _DIM), jnp.float32),
        scratch_types=[
            pltpu.VMEM((_S,), jnp.int32),
            pltpu.VMEM((_S,), jnp.int32),
            pltpu.VMEM((4, 128), jnp.int32),
            pltpu.VMEM((4, 128), jnp.int32),
            pltpu.VMEM((_S,), jnp.float32),
            pltpu.VMEM((_S,), jnp.float32),
            pltpu.VMEM((_S, _DIM), jnp.float32),
            pltpu.VMEM((_S, _DIM), jnp.float32),
            pltpu.VMEM((128,), jnp.int32),
            pltpu.VMEM_SHARED((_ACC_ROWS, _DIM), jnp.float32),
            pltpu.SemaphoreType.DMA,
            pltpu.SemaphoreType.DMA,
            pltpu.SemaphoreType.DMA,
            pltpu.SemaphoreType.DMA,
        ],
    )
    def spmm(x_hbm, srcl_hbm, dstl_hbm, wl_hbm, counts_hbm, y_hbm,
             srcA, srcB, ldstA, ldstB, wA, wB, rowsA, rowsB, cbuf, acc,
             semLA, semLB, semR, semD):
        c = lax.axis_index("c")
        s = lax.axis_index("s")
        zvec = jnp.zeros((16,), jnp.float32)
        bufs = ((srcA, ldstA, wA, rowsA, semLA), (srcB, ldstB, wB, rowsB, semLB))

        def fire_lin(off, bset):
            src_b, ldst_b, w_b, _, semL = bset
            off = pl.multiple_of(off, 128)
            pltpu.async_copy(srcl_hbm.at[pl.ds(off, _S)], src_b, semL)
            for j in range(4):
                pltpu.async_copy(dstl_hbm.at[pl.ds(off + j * 128, 128)],
                                 ldst_b.at[j], semL)
            pltpu.async_copy(wl_hbm.at[pl.ds(off, _S)], w_b, semL)

        def drain_lin(bset):
            src_b, ldst_b, w_b, _, semL = bset
            pltpu.make_async_copy(srcl_hbm.at[pl.ds(0, _S)], src_b, semL).wait()
            for j in range(4):
                pltpu.make_async_copy(dstl_hbm.at[pl.ds(0, 128)],
                                      ldst_b.at[j], semL).wait()
            pltpu.make_async_copy(wl_hbm.at[pl.ds(0, _S)], w_b, semL).wait()

        def fire_rows(bset):
            src_b, _, _, rows_b, _ = bset
            return [pltpu.async_copy(x_hbm.at[src_b.at[pl.ds(j * 128, 128)]],
                                     rows_b.at[pl.ds(j * 128, 128)], semR)
                    for j in range(4)]

        def fire_scatter(bset):
            _, ldst_b, _, rows_b, _ = bset
            return [pltpu.async_copy(rows_b.at[pl.ds(j * 128, 128)],
                                     acc.at[ldst_b.at[j]], semD, add=True)
                    for j in range(4)]

        def scale(bset):
            _, _, w_b, rows_b, _ = bset

            def scale_body(g, _):
                wv16 = w_b[pl.ds(g * 16, 16)]
                for u in range(16):
                    e = g * 16 + u
                    wv = jnp.broadcast_to(wv16[u], (16,))
                    for k in range(4):
                        rows_b[e, pl.ds(k * 16, 16)] = (
                            rows_b[e, pl.ds(k * 16, 16)] * wv)
                return 0
            lax.fori_loop(0, _S // 16, scale_body, 0)

        for p in range(2):
            qlo = c * (2 * _Q) + p * _Q
            zbase = s * (_ACC_ROWS // 16)

            def zero_rows(r, _):
                for k in range(4):
                    rowsA[r, pl.ds(k * 16, 16)] = zvec
                return 0
            lax.fori_loop(0, _S, zero_rows, 0)
            pltpu.sync_copy(rowsA, acc.at[pl.ds(pl.multiple_of(zbase, 16), _S)])
            pltpu.sync_copy(rowsA.at[pl.ds(0, 272)],
                            acc.at[pl.ds(pl.multiple_of(zbase + _S, 16), 272)])
            plsc.subcore_barrier()

            wa = s * 2
            pltpu.sync_copy(
                counts_hbm.at[pl.ds(pl.multiple_of(wa * 128, 128), 128)], cbuf)
            cvec = cbuf[pl.ds(0, 16)]
            cntA = jnp.where(c == 0, cvec[p], cvec[2 + p])
            pltpu.sync_copy(
                counts_hbm.at[pl.ds(pl.multiple_of(wa * 128 + 128, 128), 128)],
                cbuf)
            cvec = cbuf[pl.ds(0, 16)]
            cntB = jnp.where(c == 0, cvec[p], cvec[2 + p])
            nbA = (cntA + (_S - 1)) // _S
            nbB = (cntB + (_S - 1)) // _S
            q = c * 2 + p
            lbaseA = wa * (4 * _QSTRIDE) + q * _QSTRIDE
            lbaseB = lbaseA + 4 * _QSTRIDE
            npairs = jnp.maximum((nbA + nbB + 1) // 2, 1)

            def off_of(t):
                return jnp.where(t < nbA, lbaseA + t * _S,
                                 lbaseB + (t - nbA) * _S)

            fire_lin(off_of(0), bufs[0])
            fire_lin(off_of(1), bufs[1])

            def pair_body(k, _):
                t0 = k * 2
                drain_lin(bufs[0])
                rA = fire_rows(bufs[0])
                drain_lin(bufs[1])
                rB = fire_rows(bufs[1])
                for d in rA:
                    d.wait()
                scale(bufs[0])
                dA = fire_scatter(bufs[0])
                for d in rB:
                    d.wait()
                scale(bufs[1])
                dB = fire_scatter(bufs[1])
                for d in dA:
                    d.wait()
                fire_lin(off_of(t0 + 2), bufs[0])
                for d in dB:
                    d.wait()
                fire_lin(off_of(t0 + 3), bufs[1])
                return 0
            lax.fori_loop(0, npairs, pair_body, 0)
            drain_lin(bufs[0])
            drain_lin(bufs[1])

            plsc.subcore_barrier()
            pltpu.sync_copy(acc.at[pl.ds(pl.multiple_of(zbase, 16), _S)],
                            y_hbm.at[pl.ds(pl.multiple_of(qlo + zbase, 8), _S)])
            if p == 0:
                pltpu.sync_copy(
                    acc.at[pl.ds(pl.multiple_of(zbase + _S, 16), 272)],
                    y_hbm.at[pl.ds(pl.multiple_of(qlo + zbase + _S, 8), 272)])
            else:
                last = jnp.logical_and(c == 1, s == 15)

                @pl.when(jnp.logical_not(last))
                def _():
                    pltpu.sync_copy(
                        acc.at[pl.ds(pl.multiple_of(zbase + _S, 16), 272)],
                        y_hbm.at[pl.ds(pl.multiple_of(qlo + zbase + _S, 8), 272)])

                @pl.when(last)
                def _():
                    pltpu.sync_copy(
                        acc.at[pl.ds(pl.multiple_of(zbase + _S, 16), 96)],
                        y_hbm.at[pl.ds(pl.multiple_of(qlo + zbase + _S, 8), 96)])
            if p == 0:
                plsc.subcore_barrier()
    return spmm


_spmm_k = _make_spmm()


@functools.partial(
    pl.kernel,
    mesh=_mesh,
    compiler_params=pltpu.CompilerParams(use_tc_tiling_on_sc=False),
    out_type=jax.ShapeDtypeStruct((_BATCH, _DIM), jnp.float32),
    scratch_types=[
        pltpu.VMEM((256,), jnp.int32),
        pltpu.VMEM((256, _DIM), jnp.float32),
        pltpu.VMEM((256, _DIM), jnp.float32),
        pltpu.SemaphoreType.DMA,
    ],
)
def _users_mean_k(users_hbm, ue_hbm, y1, y2, y3, y4, out_hbm,
                  uidx, gbuf, abuf, sem):
    c = lax.axis_index("c")
    s = lax.axis_index("s")
    wid = s * 2 + c

    @pl.when(wid < 4)
    def _():
        ubase = pl.multiple_of(wid * 256, 256)
        pltpu.sync_copy(users_hbm.at[pl.ds(ubase, 256)], uidx)
        for j in range(2):
            pltpu.async_copy(ue_hbm.at[uidx.at[pl.ds(j * 128, 128)]],
                             abuf.at[pl.ds(j * 128, 128)], sem).wait()
        for yk in (y1, y2, y3, y4):
            for j in range(2):
                pltpu.async_copy(yk.at[uidx.at[pl.ds(j * 128, 128)]],
                                 gbuf.at[pl.ds(j * 128, 128)], sem).wait()
            def add_body(r, _):
                for k in range(4):
                    abuf[r, pl.ds(k * 16, 16)] = (abuf[r, pl.ds(k * 16, 16)]
                                                  + gbuf[r, pl.ds(k * 16, 16)])
                return 0
            lax.fori_loop(0, 256, add_body, 0)
        def fin_body(r, _):
            for k in range(4):
                abuf[r, pl.ds(k * 16, 16)] = abuf[r, pl.ds(k * 16, 16)] * 0.2
            return 0
        lax.fori_loop(0, 256, fin_body, 0)
        pltpu.sync_copy(abuf, out_hbm.at[pl.ds(ubase, 256)])


_BI = 1280
_GRID_I = 20


def _rating_body(u_ref, e_ref, a_ref, b_ref, c_ref, d_ref, o_ref):
    items = (e_ref[...] + a_ref[...] + b_ref[...] + c_ref[...] + d_ref[...]) * 0.2
    acc = lax.dot_general(u_ref[...], items, (((1,), (1,)), ((), ())),
                          preferred_element_type=jnp.float32)
    o_ref[...] = jax.nn.sigmoid(acc)


def _rating_tc(um, ie, i1, i2, i3, i4):
    bs_items = pl.BlockSpec((_BI, _DIM), lambda j: (j, 0))
    return pl.pallas_call(
        _rating_body,
        grid=(_GRID_I,),
        in_specs=[pl.BlockSpec((_BATCH, _DIM), lambda j: (0, 0))] + [bs_items] * 5,
        out_specs=pl.BlockSpec((_BATCH, _BI), lambda j: (0, j)),
        out_shape=jax.ShapeDtypeStruct((_BATCH, _N_ITEMS), jnp.float32),
    )(um, ie, i1, i2, i3, i4)


def kernel(users, edge_index, edge_weight, user_emb, item_emb):
    users = users.astype(jnp.int32)
    src = edge_index[0].astype(jnp.int32)
    dst = edge_index[1].astype(jnp.int32)
    w = edge_weight.astype(jnp.float32)
    x0 = jnp.concatenate([user_emb, item_emb], axis=0)
    srcl, dstl, wl, counts = _partition_k(src, dst, w)
    y1 = _spmm_k(x0, srcl, dstl, wl, counts)
    y2 = _spmm_k(y1, srcl, dstl, wl, counts)
    y3 = _spmm_k(y2, srcl, dstl, wl, counts)
    y4 = _spmm_k(y3, srcl, dstl, wl, counts)
    um = _users_mean_k(users, user_emb, y1, y2, y3, y4)
    items = [item_emb] + [
        lax.slice(yk, (_N_USERS, 0), (_N_NODES, _DIM)) for yk in (y1, y2, y3, y4)
    ]
    return _rating_tc(um, *items)


# speedup vs baseline: 1.6294x; 1.6294x over previous
import functools

import jax
import jax.numpy as jnp
from jax import lax
from jax.experimental import pallas as pl
from jax.experimental.pallas import tpu as pltpu
from jax.experimental.pallas import tpu_sc as plsc

_N_USERS = 25000
_N_ITEMS = 25000
_N_NODES = 50000
_E = 800000
_DIM = 64
_BATCH = 1024

_Q = 12544
_ACC_ROWS = 12608
_DUMMY = 12600
_CHUNK = 3200
_NCHT = _E // _CHUNK
_QSTRIDE = 27648
_LISTS = 32 * 4 * _QSTRIDE
_S = 512

_mesh = plsc.VectorSubcoreMesh(core_axis_name="c", subcore_axis_name="s")


def _prefix16(x, iota):
    for d in (1, 2, 4, 8):
        sh = x[jnp.maximum(iota - d, 0)]
        x = x + jnp.where(iota >= d, sh, 0)
    return x


@functools.partial(
    pl.kernel,
    mesh=_mesh,
    compiler_params=pltpu.CompilerParams(use_tc_tiling_on_sc=False),
    out_type=(
        jax.ShapeDtypeStruct((_LISTS,), jnp.int32),
        jax.ShapeDtypeStruct((4096,), jnp.int32),
    ),
    scratch_types=[
        pltpu.VMEM((_CHUNK,), jnp.int32),
        pltpu.VMEM((_CHUNK,), jnp.int32),
        pltpu.VMEM((25, 128), jnp.int32),
        pltpu.VMEM((12, 128), jnp.int32),
        pltpu.VMEM((1536,), jnp.int32),
        pltpu.SemaphoreType.DMA,
    ],
)
def _partition_k(dst_hbm, lists_hbm, counts_hbm,
                 dst_scan, ramp, pos_buf, tpos_buf, tv_buf, sem):
    c = lax.axis_index("c")
    s = lax.axis_index("s")
    w = s * 2 + c
    iota = lax.iota(jnp.int32, 16)
    base0 = w * (4 * _QSTRIDE)

    nch = jnp.where(w < _NCHT - (_NCHT // 32) * 32, _NCHT // 32 + 1, _NCHT // 32)

    def chunk_body(r, cnts):
        ch = r * 32 + w
        eoff = pl.multiple_of(ch * _CHUNK, 128)
        pltpu.sync_copy(dst_hbm.at[pl.ds(eoff, _CHUNK)], dst_scan)

        def vec_body(ii, u, cnts):
            c0, c1, c2, c3 = cnts
            boff = ii * 128 + u * 16
            v = dst_scan[pl.ds(boff, 16)]
            m1 = (v >= _Q) & (v < 2 * _Q)
            m2 = (v >= 2 * _Q) & (v < 3 * _Q)
            m3 = v >= 3 * _Q
            p0 = _prefix16(jnp.where(v < _Q, 1, 0), iota)
            p1 = _prefix16(jnp.where(m1, 1, 0), iota)
            p2 = _prefix16(jnp.where(m2, 1, 0), iota)
            p3 = _prefix16(jnp.where(m3, 1, 0), iota)
            pos = base0 + c0 + p0 - 1
            pos = jnp.where(m1, base0 + _QSTRIDE + c1 + p1 - 1, pos)
            pos = jnp.where(m2, base0 + 2 * _QSTRIDE + c2 + p2 - 1, pos)
            pos = jnp.where(m3, base0 + 3 * _QSTRIDE + c3 + p3 - 1, pos)
            pos_buf[ii, pl.ds(u * 16, 16)] = pos
            ramp[pl.ds(boff, 16)] = eoff + boff + iota
            return (c0 + p0[15], c1 + p1[15], c2 + p2[15], c3 + p3[15])

        def oct_body(ii, cnts):
            for u in range(8):
                cnts = vec_body(ii, u, cnts)
            return cnts
        cnts = lax.fori_loop(0, _CHUNK // 128, oct_body, cnts)
        descs = [pltpu.async_copy(ramp.at[pl.ds(j * 128, 128)],
                                  lists_hbm.at[pos_buf.at[j]], sem)
                 for j in range(25)]
        for d in descs:
            d.wait()
        return cnts

    z = jnp.int32(0)
    cn = lax.fori_loop(0, nch, chunk_body, (z, z, z, z))

    def tv_body(i, _):
        tv_buf[pl.ds(i * 16, 16)] = jnp.zeros((16,), jnp.int32)
        return 0
    lax.fori_loop(0, 1536 // 16, tv_body, 0)
    for qi in range(4):
        tb = base0 + qi * _QSTRIDE + cn[qi]
        for i in range(96):
            tpos_buf[i // 8, pl.ds((i % 8) * 16, 16)] = tb + i * 16 + iota
        descs = [pltpu.async_copy(tv_buf.at[pl.ds(j * 128, 128)],
                                  lists_hbm.at[tpos_buf.at[j]], sem)
                 for j in range(12)]
        for d in descs:
            d.wait()

    cv = jnp.where(iota == 0, cn[0], jnp.where(iota == 1, cn[1],
                   jnp.where(iota == 2, cn[2], jnp.where(iota == 3, cn[3], 0))))
    tv_buf[pl.ds(0, 16)] = cv
    for u in range(1, 8):
        tv_buf[pl.ds(u * 16, 16)] = jnp.zeros((16,), jnp.int32)
    pltpu.sync_copy(tv_buf.at[pl.ds(0, 128)],
                    counts_hbm.at[pl.ds(pl.multiple_of(w * 128, 128), 128)])


def _make_spmm():
    @functools.partial(
        pl.kernel,
        mesh=_mesh,
        compiler_params=pltpu.CompilerParams(use_tc_tiling_on_sc=False),
        out_type=jax.ShapeDtypeStruct((_N_NODES, _DIM), jnp.float32),
        scratch_types=[
            pltpu.VMEM((_S,), jnp.int32),
            pltpu.VMEM((_S,), jnp.int32),
            pltpu.VMEM((_S,), jnp.int32),
            pltpu.VMEM((_S,), jnp.int32),
            pltpu.VMEM((_S,), jnp.int32),
            pltpu.VMEM((_S,), jnp.int32),
            pltpu.VMEM((4, 128), jnp.int32),
            pltpu.VMEM((4, 128), jnp.int32),
            pltpu.VMEM((_S,), jnp.float32),
            pltpu.VMEM((_S,), jnp.float32),
            pltpu.VMEM((_S, _DIM), jnp.float32),
            pltpu.VMEM((_S, _DIM), jnp.float32),
            pltpu.VMEM((128,), jnp.int32),
            pltpu.VMEM_SHARED((_ACC_ROWS, _DIM), jnp.float32),
            pltpu.SemaphoreType.DMA,
            pltpu.SemaphoreType.DMA,
            pltpu.SemaphoreType.DMA,
            pltpu.SemaphoreType.DMA,
            pltpu.SemaphoreType.DMA,
            pltpu.SemaphoreType.DMA,
            pltpu.SemaphoreType.DMA,
            pltpu.SemaphoreType.DMA,
        ],
    )
    def spmm(x_hbm, src_hbm, dst_hbm, w_hbm, lists_hbm, counts_hbm, y_hbm,
             eidA, eidB, srcA, srcB, dstA, dstB, ldstA, ldstB, wA, wB,
             rowsA, rowsB, cbuf, acc,
             semLA, semLB, semIA, semIB, semRA, semRB, semDA, semDB):
        c = lax.axis_index("c")
        s = lax.axis_index("s")
        iota = lax.iota(jnp.int32, 16)
        zvec = jnp.zeros((16,), jnp.float32)
        bufs = ((eidA, srcA, dstA, ldstA, wA, rowsA, semLA, semIA, semRA, semDA),
                (eidB, srcB, dstB, ldstB, wB, rowsB, semLB, semIB, semRB, semDB))

        def fire_lin(off, bset):
            eid_b, semL = bset[0], bset[6]
            off = pl.multiple_of(off, 128)
            pltpu.async_copy(lists_hbm.at[pl.ds(off, _S)], eid_b, semL)

        def drain_lin(bset):
            eid_b, semL = bset[0], bset[6]
            pltpu.make_async_copy(lists_hbm.at[pl.ds(0, _S)], eid_b, semL).wait()

        def fire_idx(bset):
            eid_b, src_b, dst_b, w_b, semI = (bset[0], bset[1], bset[2],
                                              bset[4], bset[7])
            descs = []
            for j in range(4):
                idx = eid_b.at[pl.ds(j * 128, 128)]
                descs.append(pltpu.async_copy(
                    src_hbm.at[idx], src_b.at[pl.ds(j * 128, 128)], semI))
                descs.append(pltpu.async_copy(
                    dst_hbm.at[idx], dst_b.at[pl.ds(j * 128, 128)], semI))
                descs.append(pltpu.async_copy(
                    w_hbm.at[idx], w_b.at[pl.ds(j * 128, 128)], semI))
            return descs

        def fire_rows(bset):
            src_b, rows_b, semR = bset[1], bset[5], bset[8]
            return [pltpu.async_copy(x_hbm.at[src_b.at[pl.ds(j * 128, 128)]],
                                     rows_b.at[pl.ds(j * 128, 128)], semR)
                    for j in range(4)]

        def fire_scatter(bset):
            ldst_b, rows_b, semD = bset[3], bset[5], bset[9]
            return [pltpu.async_copy(rows_b.at[pl.ds(j * 128, 128)],
                                     acc.at[ldst_b.at[j]], semD, add=True)
                    for j in range(4)]

        for p in range(2):
            qlo = c * (2 * _Q) + p * _Q
            zbase = s * 784

            def zero_rows(r, _):
                for k in range(4):
                    rowsA[r, pl.ds(k * 16, 16)] = zvec
                return 0
            lax.fori_loop(0, _S, zero_rows, 0)
            pltpu.sync_copy(rowsA, acc.at[pl.ds(pl.multiple_of(zbase, 16), _S)])
            pltpu.sync_copy(rowsA.at[pl.ds(0, 272)],
                            acc.at[pl.ds(pl.multiple_of(zbase + _S, 16), 272)])
            plsc.subcore_barrier()

            wa = s * 2
            pltpu.sync_copy(
                counts_hbm.at[pl.ds(pl.multiple_of(wa * 128, 128), 128)], cbuf)
            cvec = cbuf[pl.ds(0, 16)]
            cntA = jnp.where(c == 0, cvec[p], cvec[2 + p])
            pltpu.sync_copy(
                counts_hbm.at[pl.ds(pl.multiple_of(wa * 128 + 128, 128), 128)],
                cbuf)
            cvec = cbuf[pl.ds(0, 16)]
            cntB = jnp.where(c == 0, cvec[p], cvec[2 + p])
            nbA = (cntA + (_S - 1)) // _S
            nbB = (cntB + (_S - 1)) // _S
            q = c * 2 + p
            lbaseA = wa * (4 * _QSTRIDE) + q * _QSTRIDE
            lbaseB = lbaseA + 4 * _QSTRIDE
            nbt = nbA + nbB
            npairs = jnp.maximum((nbt + 1) // 2, 1)

            def off_of(t):
                return jnp.where(t < nbA, lbaseA + t * _S,
                                 lbaseB + (t - nbA) * _S)

            def cnt_of(t):
                return jnp.where(t < nbA, cntA - t * _S, cntB - (t - nbA) * _S)

            def masks(t, bset):
                dst_b, ldst_b, w_b = bset[2], bset[3], bset[4]
                cnt = cnt_of(t)
                dummy = _DUMMY + (s & 7)
                for tt in range(_S // 16):
                    v = dst_b[pl.ds(tt * 16, 16)]
                    valid = (tt * 16 + iota) < cnt
                    ldst_b[tt // 8, pl.ds((tt % 8) * 16, 16)] = (
                        jnp.where(valid, v - qlo, dummy))
                    wv = w_b[pl.ds(tt * 16, 16)]
                    w_b[pl.ds(tt * 16, 16)] = jnp.where(valid, wv, 0.0)

            def scale(bset):
                w_b, rows_b = bset[4], bset[5]

                def scale_body(g, _):
                    wv16 = w_b[pl.ds(g * 16, 16)]
                    for u in range(16):
                        e = g * 16 + u
                        wv = jnp.broadcast_to(wv16[u], (16,))
                        for k in range(4):
                            rows_b[e, pl.ds(k * 16, 16)] = (
                                rows_b[e, pl.ds(k * 16, 16)] * wv)
                    return 0
                lax.fori_loop(0, _S // 16, scale_body, 0)

            fire_lin(off_of(0), bufs[0])
            fire_lin(off_of(1), bufs[1])

            def pair_body(k, _):
                t0 = k * 2
                drain_lin(bufs[0])
                iA = fire_idx(bufs[0])
                drain_lin(bufs[1])
                iB = fire_idx(bufs[1])
                for d in iA:
                    d.wait()
                rA = fire_rows(bufs[0])
                masks(t0, bufs[0])
                for d in iB:
                    d.wait()
                rB = fire_rows(bufs[1])
                masks(t0 + 1, bufs[1])
                for d in rA:
                    d.wait()
                scale(bufs[0])
                dA = fire_scatter(bufs[0])
                for d in rB:
                    d.wait()
                scale(bufs[1])
                dB = fire_scatter(bufs[1])
                for d in dA:
                    d.wait()
                fire_lin(off_of(t0 + 2), bufs[0])
                for d in dB:
                    d.wait()
                fire_lin(off_of(t0 + 3), bufs[1])
                return 0
            lax.fori_loop(0, npairs, pair_body, 0)
            drain_lin(bufs[0])
            drain_lin(bufs[1])

            plsc.subcore_barrier()
            pltpu.sync_copy(acc.at[pl.ds(pl.multiple_of(zbase, 16), _S)],
                            y_hbm.at[pl.ds(pl.multiple_of(qlo + zbase, 8), _S)])
            if p == 0:
                pltpu.sync_copy(
                    acc.at[pl.ds(pl.multiple_of(zbase + _S, 16), 272)],
                    y_hbm.at[pl.ds(pl.multiple_of(qlo + zbase + _S, 8), 272)])
            else:
                last = jnp.logical_and(c == 1, s == 15)

                @pl.when(jnp.logical_not(last))
                def _():
                    pltpu.sync_copy(
                        acc.at[pl.ds(pl.multiple_of(zbase + _S, 16), 272)],
                        y_hbm.at[pl.ds(pl.multiple_of(qlo + zbase + _S, 8), 272)])

                @pl.when(last)
                def _():
                    pltpu.sync_copy(
                        acc.at[pl.ds(pl.multiple_of(zbase + _S, 16), 96)],
                        y_hbm.at[pl.ds(pl.multiple_of(qlo + zbase + _S, 8), 96)])
            if p == 0:
                plsc.subcore_barrier()
    return spmm


_spmm_k = _make_spmm()


@functools.partial(
    pl.kernel,
    mesh=_mesh,
    compiler_params=pltpu.CompilerParams(use_tc_tiling_on_sc=False),
    out_type=jax.ShapeDtypeStruct((_BATCH, _DIM), jnp.float32),
    scratch_types=[
        pltpu.VMEM((256,), jnp.int32),
        pltpu.VMEM((256, _DIM), jnp.float32),
        pltpu.VMEM((256, _DIM), jnp.float32),
        pltpu.SemaphoreType.DMA,
    ],
)
def _users_mean_k(users_hbm, ue_hbm, y1, y2, y3, y4, out_hbm,
                  uidx, gbuf, abuf, sem):
    c = lax.axis_index("c")
    s = lax.axis_index("s")
    wid = s * 2 + c

    @pl.when(wid < 4)
    def _():
        ubase = pl.multiple_of(wid * 256, 256)
        pltpu.sync_copy(users_hbm.at[pl.ds(ubase, 256)], uidx)
        for j in range(2):
            pltpu.async_copy(ue_hbm.at[uidx.at[pl.ds(j * 128, 128)]],
                             abuf.at[pl.ds(j * 128, 128)], sem).wait()
        for yk in (y1, y2, y3, y4):
            for j in range(2):
                pltpu.async_copy(yk.at[uidx.at[pl.ds(j * 128, 128)]],
                                 gbuf.at[pl.ds(j * 128, 128)], sem).wait()
            def add_body(r, _):
                for k in range(4):
                    abuf[r, pl.ds(k * 16, 16)] = (abuf[r, pl.ds(k * 16, 16)]
                                                  + gbuf[r, pl.ds(k * 16, 16)])
                return 0
            lax.fori_loop(0, 256, add_body, 0)
        def fin_body(r, _):
            for k in range(4):
                abuf[r, pl.ds(k * 16, 16)] = abuf[r, pl.ds(k * 16, 16)] * 0.2
            return 0
        lax.fori_loop(0, 256, fin_body, 0)
        pltpu.sync_copy(abuf, out_hbm.at[pl.ds(ubase, 256)])


_BI = 1280
_GRID_I = 20


def _rating_body(u_ref, e_ref, a_ref, b_ref, c_ref, d_ref, o_ref):
    items = (e_ref[...] + a_ref[...] + b_ref[...] + c_ref[...] + d_ref[...]) * 0.2
    acc = lax.dot_general(u_ref[...], items, (((1,), (1,)), ((), ())),
                          preferred_element_type=jnp.float32)
    o_ref[...] = jax.nn.sigmoid(acc)


def _rating_tc(um, ie, i1, i2, i3, i4):
    bs_items = pl.BlockSpec((_BI, _DIM), lambda j: (j, 0))
    return pl.pallas_call(
        _rating_body,
        grid=(_GRID_I,),
        in_specs=[pl.BlockSpec((_BATCH, _DIM), lambda j: (0, 0))] + [bs_items] * 5,
        out_specs=pl.BlockSpec((_BATCH, _BI), lambda j: (0, j)),
        out_shape=jax.ShapeDtypeStruct((_BATCH, _N_ITEMS), jnp.float32),
    )(um, ie, i1, i2, i3, i4)


def kernel(users, edge_index, edge_weight, user_emb, item_emb):
    users = users.astype(jnp.int32)
    src = edge_index[0].astype(jnp.int32)
    dst = edge_index[1].astype(jnp.int32)
    w = edge_weight.astype(jnp.float32)
    x0 = jnp.concatenate([user_emb, item_emb], axis=0)
    eidl, counts = _partition_k(dst)
    y1 = _spmm_k(x0, src, dst, w, eidl, counts)
    y2 = _spmm_k(y1, src, dst, w, eidl, counts)
    y3 = _spmm_k(y2, src, dst, w, eidl, counts)
    y4 = _spmm_k(y3, src, dst, w, eidl, counts)
    um = _users_mean_k(users, user_emb, y1, y2, y3, y4)
    items = [item_emb] + [
        lax.slice(yk, (_N_USERS, 0), (_N_NODES, _DIM)) for yk in (y1, y2, y3, y4)
    ]
    return _rating_tc(um, *items)
